# Initial kernel scaffold; baseline (speedup 1.0000x reference)
import jax
import jax.numpy as jnp
from jax.experimental import pallas as pl


def kernel(u, eps, grid_flat, conv1_w, conv1_b, conv2_w, conv2_b, conv3_w, conv3_b, conv4_w, conv4_b, enc_l1_w, enc_l1_b, enc_l2_w, enc_l2_b, enc_l3_w, enc_l3_b, dx1_w, dx1_b, dx2_w, dx2_b, dx3_w, dx3_b, dz1_w, dz1_b, dz2_w, dz2_b, dz3_w, dz3_b, dj1_w, dj1_b, dj2_w, dj2_b, dj3_w, dj3_b):
    raise NotImplementedError("write your pallas kernel here")


# speedup vs baseline: 1.0864x; 1.0864x over previous
import functools

import jax
import jax.numpy as jnp
from jax.experimental import pallas as pl
from jax.experimental.pallas import tpu as pltpu

_LATENT = 32
_GRID_N = 48

_CP = pltpu.CompilerParams(
    dimension_semantics=("parallel",),
    vmem_limit_bytes=64 * 1024 * 1024,
)


def _gelu_tanh(x):
    c = 0.7978845608028654
    return 0.5 * x * (1.0 + jnp.tanh(c * (x + 0.044715 * x * x * x)))


def _softplus(x):
    return jnp.maximum(x, 0.0) + jnp.log(1.0 + jnp.exp(-jnp.abs(x)))


def _act(x, kind):
    if kind == "gelu":
        return _gelu_tanh(x)
    if kind == "relu":
        return jnp.maximum(x, 0.0)
    return x


def _ceil_to(n, m):
    return ((n + m - 1) // m) * m


def _mlp_body(x_ref, *refs, acts):
    o_ref = refs[-1]
    h = x_ref[...]
    for i, a in enumerate(acts):
        w = refs[2 * i][...]
        b = refs[2 * i + 1][...]
        h = jnp.dot(h, w, preferred_element_type=jnp.float32) + b
        h = _act(h, a)
    o_ref[...] = h


def _mlp(x2d, layers, acts, tile_rows):
    m, k = x2d.shape
    if k < 8:
        w0, b0 = layers[0]
        x2d = jnp.pad(x2d, ((0, 0), (0, 8 - k)))
        layers = [(jnp.pad(w0, ((0, 8 - k), (0, 0))), b0)] + list(layers[1:])
        k = 8
    tm = min(tile_rows, _ceil_to(m, 8))
    mp = _ceil_to(m, tm)
    if mp != m:
        x2d = jnp.pad(x2d, ((0, mp - m), (0, 0)))
    args = [x2d]
    specs = [pl.BlockSpec((tm, k), lambda i: (i, 0))]
    for w, b in layers:
        args += [w, b.reshape(1, -1)]
        specs += [pl.BlockSpec(w.shape, lambda i: (0, 0)),
                  pl.BlockSpec((1, w.shape[1]), lambda i: (0, 0))]
    n_out = layers[-1][0].shape[1]
    out = pl.pallas_call(
        functools.partial(_mlp_body, acts=tuple(acts)),
        out_shape=jax.ShapeDtypeStruct((mp, n_out), jnp.float32),
        grid=(mp // tm,),
        in_specs=specs,
        out_specs=pl.BlockSpec((tm, n_out), lambda i: (i, 0)),
        compiler_params=_CP,
    )(*args)
    return out[:m] if mp != m else out


def _patches_2x2(x):
    return jnp.concatenate(
        [x[:, :-1, :-1, :], x[:, :-1, 1:, :], x[:, 1:, :-1, :], x[:, 1:, 1:, :]],
        axis=-1)


def _pool2(x):
    b, h, w, c = x.shape
    x = x[:, : 2 * (h // 2), : 2 * (w // 2), :]
    return jnp.maximum(
        jnp.maximum(x[:, 0::2, 0::2, :], x[:, 0::2, 1::2, :]),
        jnp.maximum(x[:, 1::2, 0::2, :], x[:, 1::2, 1::2, :]))


def _conv_gelu(x, w, b, tile_rows):
    bsz, h, wd, c = x.shape
    p = _patches_2x2(x).reshape(bsz * (h - 1) * (wd - 1), 4 * c)
    y = _mlp(p, [(w, b)], ["gelu"], tile_rows)
    return y.reshape(bsz, h - 1, wd - 1, w.shape[1])


def _joint_body(zp_ref, xp_ref, w2_ref, b2_ref, w3_ref, b3_ref, o_ref):
    xp = xp_ref[...]
    zp = zp_ref[...]
    h = jnp.maximum(xp[None, :, :] + zp[:, None, :], 0.0)
    h = h.astype(jnp.bfloat16)
    w2 = w2_ref[...]
    h2 = jax.lax.dot_general(
        h, w2, (((2,), (0,)), ((), ())),
        preferred_element_type=jnp.float32)
    h2 = jnp.maximum(h2 + b2_ref[...], 0.0)
    y = jnp.sum(h2 * w3_ref[...], axis=-1) + b3_ref[0, 0]
    o_ref[...] = _softplus(y)


def _joint(xpart, zpart, w2, b2, w3, b3, bt):
    bsz, _ = zpart.shape
    s = xpart.shape[0]
    return pl.pallas_call(
        _joint_body,
        out_shape=jax.ShapeDtypeStruct((bsz, s), jnp.float32),
        grid=(bsz // bt,),
        in_specs=[
            pl.BlockSpec((bt, 128), lambda i: (i, 0)),
            pl.BlockSpec((s, 128), lambda i: (0, 0)),
            pl.BlockSpec((128, 256), lambda i: (0, 0)),
            pl.BlockSpec((1, 256), lambda i: (0, 0)),
            pl.BlockSpec((1, 256), lambda i: (0, 0)),
            pl.BlockSpec((1, 1), lambda i: (0, 0)),
        ],
        out_specs=pl.BlockSpec((bt, s), lambda i: (i, 0)),
        compiler_params=_CP,
    )(zpart, xpart, w2.astype(jnp.bfloat16), b2.reshape(1, -1),
      w3.reshape(1, -1), b3.reshape(1, 1))


def kernel(u, eps, grid_flat,
           conv1_w, conv1_b, conv2_w, conv2_b, conv3_w, conv3_b, conv4_w, conv4_b,
           enc_l1_w, enc_l1_b, enc_l2_w, enc_l2_b, enc_l3_w, enc_l3_b,
           dx1_w, dx1_b, dx2_w, dx2_b, dx3_w, dx3_b,
           dz1_w, dz1_b, dz2_w, dz2_b, dz3_w, dz3_b,
           dj1_w, dj1_b, dj2_w, dj2_b, dj3_w, dj3_b):
    bsz = u.shape[0]

    h = _conv_gelu(u, conv1_w, conv1_b, 2048)
    h = _conv_gelu(h, conv2_w, conv2_b, 2048)
    h = _pool2(h)
    h = _conv_gelu(h, conv3_w, conv3_b, 2048)
    h = _conv_gelu(h, conv4_w, conv4_b, 1024)
    h = _pool2(h)
    h = h.reshape(bsz, -1)
    enc = _mlp(h, [(enc_l1_w, enc_l1_b), (enc_l2_w, enc_l2_b),
                   (enc_l3_w, enc_l3_b)],
               ["gelu", "gelu", "none"], 512)
    mean, logvar = enc[:, :_LATENT], enc[:, _LATENT:]
    z = mean + eps * jnp.exp(0.5 * logvar)

    w1x, w1z = dj1_w[:32], dj1_w[32:]
    xpart = _mlp(grid_flat,
                 [(dx1_w, dx1_b), (dx2_w, dx2_b), (dx3_w, dx3_b),
                  (w1x, dj1_b)],
                 ["relu", "relu", "none", "none"], 2304)
    zpart = _mlp(z,
                 [(dz1_w, dz1_b), (dz2_w, dz2_b), (dz3_w, dz3_b),
                  (w1z, jnp.zeros((128,), jnp.float32))],
                 ["relu", "relu", "none", "none"], 1024)

    up = _joint(xpart, zpart, dj2_w, dj2_b, dj3_w, dj3_b, 8)
    u_pred = up.reshape(bsz, _GRID_N, _GRID_N, 1)
    return mean, logvar, z, u_pred
